# Initial kernel scaffold; baseline (speedup 1.0000x reference)
import jax
import jax.numpy as jnp
from jax.experimental import pallas as pl


def kernel(emb, idxs):
    raise NotImplementedError("write your pallas kernel here")


# speedup vs baseline: 1.8644x; 1.8644x over previous
import functools

import jax
import jax.numpy as jnp
from jax import lax
from jax.experimental import pallas as pl
from jax.experimental.pallas import tpu as pltpu
from jax.experimental.pallas import tpu_sc as plsc

NC = 2
NS = 16
NW = NC * NS
CHUNK = 128
NBUF = 4


@functools.partial(jax.jit, static_argnums=(2, 3))
def _sc_gather(emb, idx3, n_chunks, dim):
    mesh = plsc.VectorSubcoreMesh(core_axis_name="c", subcore_axis_name="s")

    @functools.partial(
        pl.kernel,
        mesh=mesh,
        out_type=jax.ShapeDtypeStruct((NW, n_chunks, CHUNK, dim), jnp.float32),
        scratch_types=[
            pltpu.VMEM((n_chunks, CHUNK), jnp.int32),
            pltpu.VMEM((NBUF, CHUNK, dim), jnp.float32),
            pltpu.SemaphoreType.DMA((NBUF,)),
            pltpu.SemaphoreType.DMA((NBUF,)),
        ],
        compiler_params=pltpu.CompilerParams(use_tc_tiling_on_sc=False),
    )
    def k(table_hbm, idx_hbm, out_hbm, idx_v, rows_v, gsem, ssem):
        wid = lax.axis_index("s") * NC + lax.axis_index("c")
        pltpu.sync_copy(idx_hbm.at[wid], idx_v)

        def gather_desc(g, b):
            return pltpu.make_async_copy(
                table_hbm.at[idx_v.at[g]], rows_v.at[b], gsem.at[b])

        def store_desc(g, b):
            return pltpu.make_async_copy(
                rows_v.at[b], out_hbm.at[wid, g], ssem.at[b])

        gather_desc(0, 0).start()

        def body(g, _):
            b = lax.rem(g, NBUF)
            gn = g + 1
            bn = lax.rem(gn, NBUF)

            @pl.when(jnp.logical_and(gn < n_chunks, gn >= NBUF))
            def _():
                store_desc(gn - NBUF, bn).wait()

            @pl.when(gn < n_chunks)
            def _():
                gather_desc(gn, bn).start()

            gather_desc(g, b).wait()
            store_desc(g, b).start()
            return 0

        lax.fori_loop(0, n_chunks, body, 0, unroll=False)

        for j in range(NBUF):
            c = n_chunks - NBUF + j
            store_desc(c, c % NBUF).wait()

    return k(emb, idx3)


def kernel(emb, idxs):
    batch, hist = idxs.shape
    vocab, dim = emb.shape
    total = batch * hist
    assert total % (NW * CHUNK) == 0
    n_chunks = total // (NW * CHUNK)
    idx3 = idxs.astype(jnp.int32).reshape(NW, n_chunks, CHUNK)
    out = _sc_gather(emb, idx3, n_chunks, dim)
    return out.reshape(batch, hist, dim)


# speedup vs baseline: 1.8755x; 1.0060x over previous
import functools

import jax
import jax.numpy as jnp
from jax import lax
from jax.experimental import pallas as pl
from jax.experimental.pallas import tpu as pltpu
from jax.experimental.pallas import tpu_sc as plsc

NC = 2
NS = 16
NW = NC * NS
CHUNK = 128
NBUF = 8
AHEAD = 4


@functools.partial(jax.jit, static_argnums=(2, 3))
def _sc_gather(emb, idx3, n_chunks, dim):
    mesh = plsc.VectorSubcoreMesh(core_axis_name="c", subcore_axis_name="s")

    @functools.partial(
        pl.kernel,
        mesh=mesh,
        out_type=jax.ShapeDtypeStruct((NW, n_chunks, CHUNK, dim), jnp.float32),
        scratch_types=[
            pltpu.VMEM((n_chunks, CHUNK), jnp.int32),
            pltpu.VMEM((NBUF, CHUNK, dim), jnp.float32),
            pltpu.SemaphoreType.DMA((NBUF,)),
            pltpu.SemaphoreType.DMA((NBUF,)),
        ],
        compiler_params=pltpu.CompilerParams(use_tc_tiling_on_sc=False),
    )
    def k(table_hbm, idx_hbm, out_hbm, idx_v, rows_v, gsem, ssem):
        wid = lax.axis_index("s") * NC + lax.axis_index("c")
        pltpu.sync_copy(idx_hbm.at[wid], idx_v)

        def gather_desc(g, b):
            return pltpu.make_async_copy(
                table_hbm.at[idx_v.at[g]], rows_v.at[b], gsem.at[b])

        def store_desc(g, b):
            return pltpu.make_async_copy(
                rows_v.at[b], out_hbm.at[wid, g], ssem.at[b])

        for g0 in range(AHEAD):
            gather_desc(g0, g0).start()

        def body(g, _):
            b = lax.rem(g, NBUF)
            gn = g + AHEAD
            bn = lax.rem(gn, NBUF)

            @pl.when(jnp.logical_and(gn < n_chunks, gn >= NBUF))
            def _():
                store_desc(gn - NBUF, bn).wait()

            @pl.when(gn < n_chunks)
            def _():
                gather_desc(gn, bn).start()

            gather_desc(g, b).wait()
            store_desc(g, b).start()
            return 0

        lax.fori_loop(0, n_chunks, body, 0, unroll=False)

        for j in range(NBUF):
            c = n_chunks - NBUF + j
            store_desc(c, c % NBUF).wait()

    return k(emb, idx3)


def kernel(emb, idxs):
    batch, hist = idxs.shape
    vocab, dim = emb.shape
    total = batch * hist
    assert total % (NW * CHUNK) == 0
    n_chunks = total // (NW * CHUNK)
    idx3 = idxs.astype(jnp.int32).reshape(NW, n_chunks, CHUNK)
    out = _sc_gather(emb, idx3, n_chunks, dim)
    return out.reshape(batch, hist, dim)


# speedup vs baseline: 1.9278x; 1.0279x over previous
import functools

import jax
import jax.numpy as jnp
from jax import lax
from jax.experimental import pallas as pl
from jax.experimental.pallas import tpu as pltpu
from jax.experimental.pallas import tpu_sc as plsc

NC = 2
NS = 16
NW = NC * NS
NBUF = 4
AHEAD = 3
SB = 2
LANES = 16


@functools.partial(jax.jit, static_argnums=(2, 3, 4))
def _sc_embed(embp, idx3, batch, hist, dim):
    bpw = idx3.shape[1]
    mesh = plsc.VectorSubcoreMesh(core_axis_name="c", subcore_axis_name="s")

    @functools.partial(
        pl.kernel,
        mesh=mesh,
        out_type=jax.ShapeDtypeStruct((batch, hist, dim), jnp.float32),
        scratch_types=[
            pltpu.VMEM((bpw, hist), jnp.int32),
            pltpu.VMEM((NBUF, hist, 2 * dim), jnp.float32),
            pltpu.VMEM((SB, hist, dim), jnp.float32),
            pltpu.SemaphoreType.DMA((NBUF,)),
            pltpu.SemaphoreType.DMA((SB,)),
        ],
        compiler_params=pltpu.CompilerParams(use_tc_tiling_on_sc=True),
    )
    def k(table_hbm, idx_hbm, out_hbm, idx_v, rows_v, sbuf, gsem, ssem):
        wid = lax.axis_index("s") * NC + lax.axis_index("c")
        pltpu.sync_copy(idx_hbm.at[wid], idx_v)

        def gather_desc(g, b):
            return pltpu.make_async_copy(
                table_hbm.at[idx_v.at[g]], rows_v.at[b], gsem.at[b])

        def store_desc(g, sb):
            return pltpu.make_async_copy(
                sbuf.at[sb], out_hbm.at[wid * bpw + g], ssem.at[sb])

        for g0 in range(AHEAD):
            gather_desc(g0, g0).start()

        def compact(b, sb):
            def row(h, _):
                for l in range(dim // LANES):
                    sbuf[sb, h, pl.ds(l * LANES, LANES)] = (
                        rows_v[b, h, pl.ds(l * LANES, LANES)])
                return 0

            lax.fori_loop(0, hist, row, 0, unroll=2)

        def body(g, _):
            b = lax.rem(g, NBUF)
            gn = g + AHEAD
            bn = lax.rem(gn, NBUF)

            @pl.when(gn < bpw)
            def _():
                gather_desc(gn, bn).start()

            gather_desc(g, b).wait()

            sb = lax.rem(g, SB)

            @pl.when(g >= SB)
            def _():
                store_desc(g - SB, sb).wait()

            compact(b, sb)
            store_desc(g, sb).start()
            return 0

        lax.fori_loop(0, bpw, body, 0, unroll=False)

        for c in range(bpw - SB, bpw):
            store_desc(c, c % SB).wait()

    return k(embp, idx3)


def kernel(emb, idxs):
    batch, hist = idxs.shape
    vocab, dim = emb.shape
    bpw = batch // NW
    embp = jnp.pad(emb, ((0, 0), (0, dim)))
    idx3 = idxs.astype(jnp.int32).reshape(NW, bpw, hist)
    return _sc_embed(embp, idx3, batch, hist, dim)
